# Initial kernel scaffold; baseline (speedup 1.0000x reference)
import jax
import jax.numpy as jnp
from jax.experimental import pallas as pl


def kernel(support, support_ys, query):
    raise NotImplementedError("write your pallas kernel here")


# speedup vs baseline: 4.5819x; 4.5819x over previous
import functools

import jax
import jax.numpy as jnp
from jax.experimental import pallas as pl

_NW = 20
_NWP = 32
_KP1 = 21
_ALPHA = 0.8
_BLK = 128
_CHEB_ITERS = 40


def _thresh_kernel(g_ref, thr_ref):
    rem = jnp.maximum(g_ref[...], 0.0)

    def body(_, carry):
        rem, cnt, thr = carry
        m = jnp.max(rem, axis=1, keepdims=True)
        eq = rem == m
        take = cnt < _KP1
        thr = jnp.where(take, m, thr)
        cnt = cnt + jnp.sum(eq.astype(jnp.int32), axis=1, keepdims=True)
        rem = jnp.where(eq, -1.0, rem)
        return rem, cnt, thr

    cnt0 = jnp.zeros((rem.shape[0], 1), jnp.int32)
    thr0 = jnp.zeros((rem.shape[0], 1), jnp.float32)
    _, _, thr = jax.lax.fori_loop(0, _KP1, body, (rem, cnt0, thr0))
    thr_ref[...] = thr


def _affinity_kernel(g_ref, gt_ref, thr_ref, thrt_ref, w_ref, s_ref):
    i = pl.program_id(0)
    g = g_ref[...]
    gt = gt_ref[...]
    thr_i = thr_ref[...]
    thr_j = thrt_ref[...]
    r = jnp.maximum(g, 0.0)
    rt = jnp.maximum(gt, 0.0)
    w = ((g >= thr_i).astype(jnp.float32) * (r * r * r)
         + (gt >= thr_j).astype(jnp.float32) * (rt * rt * rt))
    rows = i * _BLK + jax.lax.broadcasted_iota(jnp.int32, g.shape, 0)
    cols = jax.lax.broadcasted_iota(jnp.int32, g.shape, 1)
    w = jnp.where(rows == cols, 0.0, w)
    w_ref[...] = w
    s_ref[...] = jnp.sum(w, axis=1, keepdims=True)


def _solve_kernel(w_ref, s_ref, ys_ref, probs_ref, lab_ref, zam_ref):
    s = s_ref[...]
    dv = jax.lax.rsqrt(jnp.where(s == 0.0, 1.0, s))
    ys = ys_ref[...]
    n_rows = s.shape[0]
    col = jax.lax.broadcasted_iota(jnp.int32, (n_rows, _NWP), 1)
    y = (col == ys).astype(jnp.float32)

    def matvec(d):
        t = dv * d
        u = jnp.dot(w_ref[...], t, preferred_element_type=jnp.float32,
                    precision=jax.lax.Precision.HIGHEST)
        return d - _ALPHA * (dv * u)

    theta, delta = 1.0, _ALPHA
    sigma1 = theta / delta

    x = jnp.zeros_like(y)
    r = y
    d = r / theta
    rho = 1.0 / sigma1

    def body(_, carry):
        x, r, d, rho = carry
        x = x + d
        r = r - matvec(d)
        rho_new = 1.0 / (2.0 * sigma1 - rho)
        d = rho_new * rho * d + (2.0 * rho_new / delta) * r
        return x, r, d, rho_new

    x, _, _, _ = jax.lax.fori_loop(0, _CHEB_ITERS, body, (x, r, d, rho))

    z = jnp.maximum(x, 0.0)
    zmax = jnp.max(z, axis=1, keepdims=True)
    zam_ref[...] = -zmax
    denom = jnp.maximum(jnp.sum(z, axis=1, keepdims=True), 1e-12)
    probs_ref[...] = z / denom
    cand = jnp.where(z == zmax, col, _NWP)
    lab_ref[...] = jnp.min(cand, axis=1, keepdims=True)


@jax.jit
def kernel(support, support_ys, query):
    ns, dfeat = support.shape
    nq = query.shape[0]
    n = ns + nq
    nblk = (n + _BLK - 1) // _BLK
    np_ = nblk * _BLK

    x_exact = jnp.concatenate([support, query], axis=0)
    ip = x_exact @ x_exact.T

    g = jnp.zeros((np_, np_), jnp.float32).at[:n, :n].set(ip)

    thr = pl.pallas_call(
        _thresh_kernel,
        grid=(nblk,),
        in_specs=[pl.BlockSpec((_BLK, np_), lambda i: (i, 0))],
        out_specs=pl.BlockSpec((_BLK, 1), lambda i: (i, 0)),
        out_shape=jax.ShapeDtypeStruct((np_, 1), jnp.float32),
    )(g)

    thrt = thr.reshape(1, np_)
    gt = g.T

    w, s = pl.pallas_call(
        _affinity_kernel,
        grid=(nblk,),
        in_specs=[
            pl.BlockSpec((_BLK, np_), lambda i: (i, 0)),
            pl.BlockSpec((_BLK, np_), lambda i: (i, 0)),
            pl.BlockSpec((_BLK, 1), lambda i: (i, 0)),
            pl.BlockSpec((1, np_), lambda i: (0, 0)),
        ],
        out_specs=[
            pl.BlockSpec((_BLK, np_), lambda i: (i, 0)),
            pl.BlockSpec((_BLK, 1), lambda i: (i, 0)),
        ],
        out_shape=[
            jax.ShapeDtypeStruct((np_, np_), jnp.float32),
            jax.ShapeDtypeStruct((np_, 1), jnp.float32),
        ],
    )(g, gt, thr, thrt)

    ys_full = jnp.full((np_, 1), -1, jnp.int32).at[:ns, 0].set(support_ys)

    probs, labels, zam = pl.pallas_call(
        _solve_kernel,
        grid=(1,),
        in_specs=[
            pl.BlockSpec((np_, np_), lambda i: (0, 0)),
            pl.BlockSpec((np_, 1), lambda i: (0, 0)),
            pl.BlockSpec((np_, 1), lambda i: (0, 0)),
        ],
        out_specs=[
            pl.BlockSpec((np_, _NWP), lambda i: (0, 0)),
            pl.BlockSpec((np_, 1), lambda i: (0, 0)),
            pl.BlockSpec((np_, 1), lambda i: (0, 0)),
        ],
        out_shape=[
            jax.ShapeDtypeStruct((np_, _NWP), jnp.float32),
            jax.ShapeDtypeStruct((np_, 1), jnp.int32),
            jax.ShapeDtypeStruct((np_, 1), jnp.float32),
        ],
    )(w, s, ys_full)

    p_labels = labels[ns:n, 0]
    probs_out = probs[ns:n, :_NW]
    z_amax = zam[ns:n, 0]
    return p_labels, probs_out, z_amax


# speedup vs baseline: 7.6332x; 1.6659x over previous
import functools

import jax
import jax.numpy as jnp
from jax.experimental import pallas as pl

_NW = 20
_NWP = 32
_KP1 = 21
_ALPHA = 0.8
_BLK = 128
_IR_OUTER = 4
_IR_INNER = 12


def _thresh_kernel(g_ref, thr_ref):
    rem = jnp.maximum(g_ref[...], 0.0)

    def body(_, carry):
        rem, cnt, thr = carry
        m = jnp.max(rem, axis=1, keepdims=True)
        eq = rem == m
        take = cnt < _KP1
        thr = jnp.where(take, m, thr)
        cnt = cnt + jnp.sum(eq.astype(jnp.int32), axis=1, keepdims=True)
        rem = jnp.where(eq, -1.0, rem)
        return rem, cnt, thr

    cnt0 = jnp.zeros((rem.shape[0], 1), jnp.int32)
    thr0 = jnp.zeros((rem.shape[0], 1), jnp.float32)
    _, _, thr = jax.lax.fori_loop(0, _KP1, body, (rem, cnt0, thr0))
    thr_ref[...] = thr


def _affinity_kernel(g_ref, gt_ref, thr_ref, thrt_ref, w0_ref, w1_ref,
                     w2_ref, s_ref):
    i = pl.program_id(0)
    g = g_ref[...]
    gt = gt_ref[...]
    thr_i = thr_ref[...]
    thr_j = thrt_ref[...]
    r = jnp.maximum(g, 0.0)
    rt = jnp.maximum(gt, 0.0)
    w = ((g >= thr_i).astype(jnp.float32) * (r * r * r)
         + (gt >= thr_j).astype(jnp.float32) * (rt * rt * rt))
    rows = i * _BLK + jax.lax.broadcasted_iota(jnp.int32, g.shape, 0)
    cols = jax.lax.broadcasted_iota(jnp.int32, g.shape, 1)
    w = jnp.where(rows == cols, 0.0, w)
    w0 = w.astype(jnp.bfloat16)
    r1 = w - w0.astype(jnp.float32)
    w1 = r1.astype(jnp.bfloat16)
    r2 = r1 - w1.astype(jnp.float32)
    w0_ref[...] = w0
    w1_ref[...] = w1
    w2_ref[...] = r2.astype(jnp.bfloat16)
    s_ref[...] = jnp.sum(w, axis=1, keepdims=True)


def _solve_kernel(w0_ref, w1_ref, w2_ref, s_ref, ys_ref, probs_ref, lab_ref,
                  zam_ref):
    s = s_ref[...]
    dv = jax.lax.rsqrt(jnp.where(s == 0.0, 1.0, s))
    ys = ys_ref[...]
    n_rows = s.shape[0]
    col = jax.lax.broadcasted_iota(jnp.int32, (n_rows, _NWP), 1)
    y = (col == ys).astype(jnp.float32)

    def _bdot(a, b):
        return jnp.dot(a, b, preferred_element_type=jnp.float32)

    def matvec_exact(v):
        t = dv * v
        t0 = t.astype(jnp.bfloat16)
        s1 = t - t0.astype(jnp.float32)
        t1 = s1.astype(jnp.bfloat16)
        t2 = (s1 - t1.astype(jnp.float32)).astype(jnp.bfloat16)
        w0 = w0_ref[...]
        w1 = w1_ref[...]
        u = (_bdot(w0, t0) + _bdot(w0, t1) + _bdot(w1, t0)
             + _bdot(w0, t2) + _bdot(w1, t1) + _bdot(w2_ref[...], t0))
        return v - _ALPHA * (dv * u)

    def matvec_cheap(v):
        t = (dv * v).astype(jnp.bfloat16)
        u = _bdot(w0_ref[...], t)
        return v - _ALPHA * (dv * u)

    sigma1 = 1.0 / _ALPHA

    def inner_solve(rr):
        def ib(_, c):
            e, res, p, rho = c
            e = e + p
            res = res - matvec_cheap(p)
            rho_new = 1.0 / (2.0 * sigma1 - rho)
            p = rho_new * rho * p + (2.0 * rho_new / _ALPHA) * res
            return e, res, p, rho_new

        e, _, _, _ = jax.lax.fori_loop(
            0, _IR_INNER, ib, (jnp.zeros_like(rr), rr, rr, _ALPHA))
        return e

    def ob(_, x):
        r = y - matvec_exact(x)
        return x + inner_solve(r)

    x = jax.lax.fori_loop(0, _IR_OUTER, ob, jnp.zeros_like(y))

    z = jnp.maximum(x, 0.0)
    zmax = jnp.max(z, axis=1, keepdims=True)
    zam_ref[...] = -zmax
    denom = jnp.maximum(jnp.sum(z, axis=1, keepdims=True), 1e-12)
    probs_ref[...] = z / denom
    cand = jnp.where(z == zmax, col, _NWP)
    lab_ref[...] = jnp.min(cand, axis=1, keepdims=True)


@jax.jit
def kernel(support, support_ys, query):
    ns, dfeat = support.shape
    nq = query.shape[0]
    n = ns + nq
    nblk = (n + _BLK - 1) // _BLK
    np_ = nblk * _BLK

    x_exact = jnp.concatenate([support, query], axis=0)
    ip = x_exact @ x_exact.T

    g = jnp.zeros((np_, np_), jnp.float32).at[:n, :n].set(ip)

    thr = pl.pallas_call(
        _thresh_kernel,
        grid=(nblk,),
        in_specs=[pl.BlockSpec((_BLK, np_), lambda i: (i, 0))],
        out_specs=pl.BlockSpec((_BLK, 1), lambda i: (i, 0)),
        out_shape=jax.ShapeDtypeStruct((np_, 1), jnp.float32),
    )(g)

    thrt = thr.reshape(1, np_)
    gt = g.T

    w0, w1, w2, s = pl.pallas_call(
        _affinity_kernel,
        grid=(nblk,),
        in_specs=[
            pl.BlockSpec((_BLK, np_), lambda i: (i, 0)),
            pl.BlockSpec((_BLK, np_), lambda i: (i, 0)),
            pl.BlockSpec((_BLK, 1), lambda i: (i, 0)),
            pl.BlockSpec((1, np_), lambda i: (0, 0)),
        ],
        out_specs=[
            pl.BlockSpec((_BLK, np_), lambda i: (i, 0)),
            pl.BlockSpec((_BLK, np_), lambda i: (i, 0)),
            pl.BlockSpec((_BLK, np_), lambda i: (i, 0)),
            pl.BlockSpec((_BLK, 1), lambda i: (i, 0)),
        ],
        out_shape=[
            jax.ShapeDtypeStruct((np_, np_), jnp.bfloat16),
            jax.ShapeDtypeStruct((np_, np_), jnp.bfloat16),
            jax.ShapeDtypeStruct((np_, np_), jnp.bfloat16),
            jax.ShapeDtypeStruct((np_, 1), jnp.float32),
        ],
    )(g, gt, thr, thrt)

    ys_full = jnp.full((np_, 1), -1, jnp.int32).at[:ns, 0].set(support_ys)

    probs, labels, zam = pl.pallas_call(
        _solve_kernel,
        grid=(1,),
        in_specs=[
            pl.BlockSpec((np_, np_), lambda i: (0, 0)),
            pl.BlockSpec((np_, np_), lambda i: (0, 0)),
            pl.BlockSpec((np_, np_), lambda i: (0, 0)),
            pl.BlockSpec((np_, 1), lambda i: (0, 0)),
            pl.BlockSpec((np_, 1), lambda i: (0, 0)),
        ],
        out_specs=[
            pl.BlockSpec((np_, _NWP), lambda i: (0, 0)),
            pl.BlockSpec((np_, 1), lambda i: (0, 0)),
            pl.BlockSpec((np_, 1), lambda i: (0, 0)),
        ],
        out_shape=[
            jax.ShapeDtypeStruct((np_, _NWP), jnp.float32),
            jax.ShapeDtypeStruct((np_, 1), jnp.int32),
            jax.ShapeDtypeStruct((np_, 1), jnp.float32),
        ],
    )(w0, w1, w2, s, ys_full)

    p_labels = labels[ns:n, 0]
    probs_out = probs[ns:n, :_NW]
    z_amax = zam[ns:n, 0]
    return p_labels, probs_out, z_amax


# speedup vs baseline: 8.6267x; 1.1302x over previous
import functools

import jax
import jax.numpy as jnp
from jax.experimental import pallas as pl

_NW = 20
_NWP = 32
_KP1 = 21
_ALPHA = 0.8
_BLK = 128
_IR_OUTER = 3
_IR_INNER = 10


def _thresh_kernel(g_ref, thr_ref):
    rem = jnp.maximum(g_ref[...], 0.0)

    def body(_, carry):
        rem, cnt, thr = carry
        m = jnp.max(rem, axis=1, keepdims=True)
        eq = rem == m
        take = cnt < _KP1
        thr = jnp.where(take, m, thr)
        cnt = cnt + jnp.sum(eq.astype(jnp.int32), axis=1, keepdims=True)
        rem = jnp.where(eq, -1.0, rem)
        return rem, cnt, thr

    cnt0 = jnp.zeros((rem.shape[0], 1), jnp.int32)
    thr0 = jnp.zeros((rem.shape[0], 1), jnp.float32)
    _, _, thr = jax.lax.fori_loop(0, _KP1, body, (rem, cnt0, thr0))
    thr_ref[...] = thr


def _affinity_kernel(g_ref, gt_ref, thr_ref, thrt_ref, w0_ref, w1_ref,
                     w2_ref, s_ref):
    i = pl.program_id(0)
    g = g_ref[...]
    gt = gt_ref[...]
    thr_i = thr_ref[...]
    thr_j = thrt_ref[...]
    r = jnp.maximum(g, 0.0)
    rt = jnp.maximum(gt, 0.0)
    w = ((g >= thr_i).astype(jnp.float32) * (r * r * r)
         + (gt >= thr_j).astype(jnp.float32) * (rt * rt * rt))
    rows = i * _BLK + jax.lax.broadcasted_iota(jnp.int32, g.shape, 0)
    cols = jax.lax.broadcasted_iota(jnp.int32, g.shape, 1)
    w = jnp.where(rows == cols, 0.0, w)
    w0 = w.astype(jnp.bfloat16)
    r1 = w - w0.astype(jnp.float32)
    w1 = r1.astype(jnp.bfloat16)
    r2 = r1 - w1.astype(jnp.float32)
    w0_ref[...] = w0
    w1_ref[...] = w1
    w2_ref[...] = r2.astype(jnp.bfloat16)
    s_ref[...] = jnp.sum(w, axis=1, keepdims=True)


def _solve_kernel(w0_ref, w1_ref, w2_ref, s_ref, ys_ref, probs_ref, lab_ref,
                  zam_ref):
    s = s_ref[...]
    dv = jax.lax.rsqrt(jnp.where(s == 0.0, 1.0, s))
    ys = ys_ref[...]
    n_rows = s.shape[0]
    col = jax.lax.broadcasted_iota(jnp.int32, (n_rows, _NWP), 1)
    y = (col == ys).astype(jnp.float32)

    def _bdot(a, b):
        return jnp.dot(a, b, preferred_element_type=jnp.float32)

    def matvec_exact(v):
        t = dv * v
        t0 = t.astype(jnp.bfloat16)
        s1 = t - t0.astype(jnp.float32)
        t1 = s1.astype(jnp.bfloat16)
        t2 = (s1 - t1.astype(jnp.float32)).astype(jnp.bfloat16)
        w0 = w0_ref[...]
        w1 = w1_ref[...]
        u = (_bdot(w0, t0) + _bdot(w0, t1) + _bdot(w1, t0)
             + _bdot(w0, t2) + _bdot(w1, t1) + _bdot(w2_ref[...], t0))
        return v - _ALPHA * (dv * u)

    def matvec_cheap(v):
        t = (dv * v).astype(jnp.bfloat16)
        u = _bdot(w0_ref[...], t)
        return v - _ALPHA * (dv * u)

    sigma1 = 1.0 / _ALPHA

    def inner_solve(rr):
        def ib(_, c):
            e, res, p, rho = c
            e = e + p
            res = res - matvec_cheap(p)
            rho_new = 1.0 / (2.0 * sigma1 - rho)
            p = rho_new * rho * p + (2.0 * rho_new / _ALPHA) * res
            return e, res, p, rho_new

        e, _, _, _ = jax.lax.fori_loop(
            0, _IR_INNER, ib, (jnp.zeros_like(rr), rr, rr, _ALPHA))
        return e

    def ob(_, x):
        r = y - matvec_exact(x)
        return x + inner_solve(r)

    x = jax.lax.fori_loop(0, _IR_OUTER, ob, jnp.zeros_like(y))

    z = jnp.maximum(x, 0.0)
    zmax = jnp.max(z, axis=1, keepdims=True)
    zam_ref[...] = -zmax
    denom = jnp.maximum(jnp.sum(z, axis=1, keepdims=True), 1e-12)
    probs_ref[...] = z / denom
    cand = jnp.where(z == zmax, col, _NWP)
    lab_ref[...] = jnp.min(cand, axis=1, keepdims=True)


@jax.jit
def kernel(support, support_ys, query):
    ns, dfeat = support.shape
    nq = query.shape[0]
    n = ns + nq
    nblk = (n + _BLK - 1) // _BLK
    np_ = nblk * _BLK

    x_exact = jnp.concatenate([support, query], axis=0)
    ip = x_exact @ x_exact.T

    g = jnp.zeros((np_, np_), jnp.float32).at[:n, :n].set(ip)

    thr = pl.pallas_call(
        _thresh_kernel,
        grid=(nblk,),
        in_specs=[pl.BlockSpec((_BLK, np_), lambda i: (i, 0))],
        out_specs=pl.BlockSpec((_BLK, 1), lambda i: (i, 0)),
        out_shape=jax.ShapeDtypeStruct((np_, 1), jnp.float32),
    )(g)

    thrt = thr.reshape(1, np_)
    gt = g.T

    w0, w1, w2, s = pl.pallas_call(
        _affinity_kernel,
        grid=(nblk,),
        in_specs=[
            pl.BlockSpec((_BLK, np_), lambda i: (i, 0)),
            pl.BlockSpec((_BLK, np_), lambda i: (i, 0)),
            pl.BlockSpec((_BLK, 1), lambda i: (i, 0)),
            pl.BlockSpec((1, np_), lambda i: (0, 0)),
        ],
        out_specs=[
            pl.BlockSpec((_BLK, np_), lambda i: (i, 0)),
            pl.BlockSpec((_BLK, np_), lambda i: (i, 0)),
            pl.BlockSpec((_BLK, np_), lambda i: (i, 0)),
            pl.BlockSpec((_BLK, 1), lambda i: (i, 0)),
        ],
        out_shape=[
            jax.ShapeDtypeStruct((np_, np_), jnp.bfloat16),
            jax.ShapeDtypeStruct((np_, np_), jnp.bfloat16),
            jax.ShapeDtypeStruct((np_, np_), jnp.bfloat16),
            jax.ShapeDtypeStruct((np_, 1), jnp.float32),
        ],
    )(g, gt, thr, thrt)

    ys_full = jnp.full((np_, 1), -1, jnp.int32).at[:ns, 0].set(support_ys)

    probs, labels, zam = pl.pallas_call(
        _solve_kernel,
        grid=(1,),
        in_specs=[
            pl.BlockSpec((np_, np_), lambda i: (0, 0)),
            pl.BlockSpec((np_, np_), lambda i: (0, 0)),
            pl.BlockSpec((np_, np_), lambda i: (0, 0)),
            pl.BlockSpec((np_, 1), lambda i: (0, 0)),
            pl.BlockSpec((np_, 1), lambda i: (0, 0)),
        ],
        out_specs=[
            pl.BlockSpec((np_, _NWP), lambda i: (0, 0)),
            pl.BlockSpec((np_, 1), lambda i: (0, 0)),
            pl.BlockSpec((np_, 1), lambda i: (0, 0)),
        ],
        out_shape=[
            jax.ShapeDtypeStruct((np_, _NWP), jnp.float32),
            jax.ShapeDtypeStruct((np_, 1), jnp.int32),
            jax.ShapeDtypeStruct((np_, 1), jnp.float32),
        ],
    )(w0, w1, w2, s, ys_full)

    p_labels = labels[ns:n, 0]
    probs_out = probs[ns:n, :_NW]
    z_amax = zam[ns:n, 0]
    return p_labels, probs_out, z_amax


# speedup vs baseline: 8.6285x; 1.0002x over previous
import jax
import jax.numpy as jnp
from jax.experimental import pallas as pl

_NW = 20
_NWP = 32
_KP1 = 21
_ALPHA = 0.8
_BLK = 128
_IR_OUTER = 3
_IR_INNER = 10


def _thresh_kernel(g_ref, thr_ref):
    rem = jnp.maximum(g_ref[...], 0.0)

    def body(_, carry):
        rem, cnt, thr = carry
        m = jnp.max(rem, axis=1, keepdims=True)
        eq = rem == m
        take = cnt < _KP1
        thr = jnp.where(take, m, thr)
        cnt = cnt + jnp.sum(eq.astype(jnp.int32), axis=1, keepdims=True)
        rem = jnp.where(eq, -1.0, rem)
        return rem, cnt, thr

    cnt0 = jnp.zeros((rem.shape[0], 1), jnp.int32)
    thr0 = jnp.zeros((rem.shape[0], 1), jnp.float32)
    _, _, thr = jax.lax.fori_loop(0, _KP1, body, (rem, cnt0, thr0))
    thr_ref[...] = thr


def _affinity_kernel(g_ref, gt_ref, thr_ref, thrt_ref, w0_ref, w1_ref,
                     w2_ref, s_ref):
    i = pl.program_id(0)
    g = g_ref[...]
    gt = gt_ref[...]
    thr_i = thr_ref[...]
    thr_j = thrt_ref[...]
    r = jnp.maximum(g, 0.0)
    rt = jnp.maximum(gt, 0.0)
    w = ((g >= thr_i).astype(jnp.float32) * (r * r * r)
         + (gt >= thr_j).astype(jnp.float32) * (rt * rt * rt))
    rows = i * _BLK + jax.lax.broadcasted_iota(jnp.int32, g.shape, 0)
    cols = jax.lax.broadcasted_iota(jnp.int32, g.shape, 1)
    w = jnp.where(rows == cols, 0.0, w)
    w0 = w.astype(jnp.bfloat16)
    r1 = w - w0.astype(jnp.float32)
    w1 = r1.astype(jnp.bfloat16)
    r2 = r1 - w1.astype(jnp.float32)
    w0_ref[...] = w0
    w1_ref[...] = w1
    w2_ref[...] = r2.astype(jnp.bfloat16)
    s_ref[...] = jnp.sum(w, axis=1, keepdims=True)


def _solve_kernel(w0_ref, w1_ref, w2_ref, s_ref, ys_ref, probs_ref, lab_ref,
                  zam_ref):
    s = s_ref[...]
    dv = jax.lax.rsqrt(jnp.where(s == 0.0, 1.0, s))
    ys = ys_ref[...]
    n_rows = s.shape[0]
    col = jax.lax.broadcasted_iota(jnp.int32, (n_rows, _NWP), 1)
    y = (col == ys).astype(jnp.float32)

    def _bdot(a, b):
        return jnp.dot(a, b, preferred_element_type=jnp.float32)

    def matvec_exact(v):
        t = dv * v
        t0 = t.astype(jnp.bfloat16)
        s1 = t - t0.astype(jnp.float32)
        t1 = s1.astype(jnp.bfloat16)
        t2 = (s1 - t1.astype(jnp.float32)).astype(jnp.bfloat16)
        w0 = w0_ref[...]
        w1 = w1_ref[...]
        u = (_bdot(w0, t0) + _bdot(w0, t1) + _bdot(w1, t0)
             + _bdot(w0, t2) + _bdot(w1, t1) + _bdot(w2_ref[...], t0))
        return v - _ALPHA * (dv * u)

    def matvec_cheap(v):
        t = (dv * v).astype(jnp.bfloat16)
        u = _bdot(w0_ref[...], t)
        return v - _ALPHA * (dv * u)

    sigma1 = 1.0 / _ALPHA

    def inner_solve(rr):
        def ib(_, c):
            e, res, p, rho = c
            e = e + p
            res = res - matvec_cheap(p)
            rho_new = 1.0 / (2.0 * sigma1 - rho)
            p = rho_new * rho * p + (2.0 * rho_new / _ALPHA) * res
            return e, res, p, rho_new

        e, _, _, _ = jax.lax.fori_loop(
            0, _IR_INNER, ib, (jnp.zeros_like(rr), rr, rr, _ALPHA))
        return e

    def ob(_, x):
        r = y - matvec_exact(x)
        return x + inner_solve(r)

    x = jax.lax.fori_loop(0, _IR_OUTER, ob, jnp.zeros_like(y))

    z = jnp.maximum(x, 0.0)
    zmax = jnp.max(z, axis=1, keepdims=True)
    zam_ref[...] = -zmax
    denom = jnp.maximum(jnp.sum(z, axis=1, keepdims=True), 1e-12)
    probs_ref[...] = z / denom
    cand = jnp.where(z == zmax, col, _NWP)
    lab_ref[...] = jnp.min(cand, axis=1, keepdims=True)


@jax.jit
def kernel(support, support_ys, query):
    ns, dfeat = support.shape
    nq = query.shape[0]
    n = ns + nq
    nblk = (n + _BLK - 1) // _BLK
    np_ = nblk * _BLK

    x_exact = jnp.concatenate([support, query], axis=0)
    ip = x_exact @ x_exact.T

    g = jnp.zeros((np_, np_), jnp.float32).at[:n, :n].set(ip)

    thr = pl.pallas_call(
        _thresh_kernel,
        grid=(nblk,),
        in_specs=[pl.BlockSpec((_BLK, np_), lambda i: (i, 0))],
        out_specs=pl.BlockSpec((_BLK, 1), lambda i: (i, 0)),
        out_shape=jax.ShapeDtypeStruct((np_, 1), jnp.float32),
    )(g)

    thrt = thr.reshape(1, np_)
    gt = g.T

    w0, w1, w2, s = pl.pallas_call(
        _affinity_kernel,
        grid=(nblk,),
        in_specs=[
            pl.BlockSpec((_BLK, np_), lambda i: (i, 0)),
            pl.BlockSpec((_BLK, np_), lambda i: (i, 0)),
            pl.BlockSpec((_BLK, 1), lambda i: (i, 0)),
            pl.BlockSpec((1, np_), lambda i: (0, 0)),
        ],
        out_specs=[
            pl.BlockSpec((_BLK, np_), lambda i: (i, 0)),
            pl.BlockSpec((_BLK, np_), lambda i: (i, 0)),
            pl.BlockSpec((_BLK, np_), lambda i: (i, 0)),
            pl.BlockSpec((_BLK, 1), lambda i: (i, 0)),
        ],
        out_shape=[
            jax.ShapeDtypeStruct((np_, np_), jnp.bfloat16),
            jax.ShapeDtypeStruct((np_, np_), jnp.bfloat16),
            jax.ShapeDtypeStruct((np_, np_), jnp.bfloat16),
            jax.ShapeDtypeStruct((np_, 1), jnp.float32),
        ],
    )(g, gt, thr, thrt)

    ys_full = jnp.full((np_, 1), -1, jnp.int32).at[:ns, 0].set(support_ys)

    probs, labels, zam = pl.pallas_call(
        _solve_kernel,
        grid=(1,),
        in_specs=[
            pl.BlockSpec((np_, np_), lambda i: (0, 0)),
            pl.BlockSpec((np_, np_), lambda i: (0, 0)),
            pl.BlockSpec((np_, np_), lambda i: (0, 0)),
            pl.BlockSpec((np_, 1), lambda i: (0, 0)),
            pl.BlockSpec((np_, 1), lambda i: (0, 0)),
        ],
        out_specs=[
            pl.BlockSpec((np_, _NWP), lambda i: (0, 0)),
            pl.BlockSpec((np_, 1), lambda i: (0, 0)),
            pl.BlockSpec((np_, 1), lambda i: (0, 0)),
        ],
        out_shape=[
            jax.ShapeDtypeStruct((np_, _NWP), jnp.float32),
            jax.ShapeDtypeStruct((np_, 1), jnp.int32),
            jax.ShapeDtypeStruct((np_, 1), jnp.float32),
        ],
    )(w0, w1, w2, s, ys_full)

    p_labels = labels[ns:n, 0]
    probs_out = probs[ns:n, :_NW]
    z_amax = zam[ns:n, 0]
    return p_labels, probs_out, z_amax
